# Initial kernel scaffold; baseline (speedup 1.0000x reference)
import jax
import jax.numpy as jnp
from jax.experimental import pallas as pl


def kernel(x, edge_index, W1, b1, W2, b2, W3, b3):
    raise NotImplementedError("write your pallas kernel here")


# speedup vs baseline: 8.8363x; 8.8363x over previous
import functools

import jax
import jax.numpy as jnp
from jax import lax
from jax.experimental import pallas as pl
from jax.experimental.pallas import tpu as pltpu
from jax.experimental.pallas import tpu_sc as plsc

N = 10000
C = 128
E = 320000
NPAD = 10240
NC, NS, L = 2, 16, 16
NW = NC * NS
CH = 128
NCH = 80
NH = NCH // 2
EPAD = NW * NCH * CH
ZR = NPAD // NS
BR = 1024


@functools.cache
def _sc_kernels():
    mesh = plsc.VectorSubcoreMesh(
        core_axis_name="c", subcore_axis_name="s",
        num_cores=NC, num_subcores=NS)

    deg_kernel = functools.partial(
        pl.kernel,
        out_type=jax.ShapeDtypeStruct((NC, NPAD), jnp.float32),
        mesh=mesh,
        scratch_types=[
            pltpu.VMEM((NCH, CH), jnp.int32),
            pltpu.VMEM((CH,), jnp.float32),
            pltpu.VMEM_SHARED((NPAD,), jnp.float32),
        ],
    )(_deg_body)

    scatter_kernel = functools.partial(
        pl.kernel,
        out_type=jax.ShapeDtypeStruct((NC, NPAD, C), jnp.float32),
        mesh=mesh,
        scratch_types=[
            pltpu.VMEM((NH, CH), jnp.int32),
            pltpu.VMEM((NH, CH), jnp.int32),
            pltpu.VMEM((CH, C), jnp.float32),
            pltpu.VMEM((CH, C), jnp.float32),
            pltpu.VMEM_SHARED((NPAD, C), jnp.float32),
            pltpu.SemaphoreType.DMA,
            pltpu.SemaphoreType.DMA,
            pltpu.SemaphoreType.DMA,
            pltpu.SemaphoreType.DMA,
        ],
    )(_scatter_body)

    return deg_kernel, scatter_kernel


def _deg_body(col_hbm, zero1_hbm, out_hbm, colv, ones, acc):
    c = lax.axis_index("c")
    s = lax.axis_index("s")
    w = s * NC + c
    for i in range(CH // L):
        ones[pl.ds(i * L, L)] = jnp.ones((L,), jnp.float32)
    pltpu.sync_copy(zero1_hbm, acc.at[pl.ds(s * ZR, ZR)])
    pltpu.sync_copy(col_hbm.at[w], colv)
    plsc.subcore_barrier()

    def body(j, carry):
        pltpu.sync_copy(ones, acc.at[colv.at[j]], add=True)
        return carry

    lax.fori_loop(0, NCH, body, 0)
    plsc.subcore_barrier()
    pltpu.sync_copy(acc.at[pl.ds(s * ZR, ZR)], out_hbm.at[c, pl.ds(s * ZR, ZR)])


def _scatter_body(g_hbm, row_hbm, col_hbm, zero2_hbm, out_hbm,
                    rowv, colv, buf0, buf1, acc, sg0, sg1, ss0, ss1):
    c = lax.axis_index("c")
    s = lax.axis_index("s")
    w = s * NC + c
    pltpu.sync_copy(zero2_hbm, acc.at[pl.ds(s * ZR, ZR)])
    plsc.subcore_barrier()

    for half in range(2):
        pltpu.sync_copy(row_hbm.at[w, pl.ds(half * NH, NH)], rowv)
        pltpu.sync_copy(col_hbm.at[w, pl.ds(half * NH, NH)], colv)
        pltpu.async_copy(g_hbm.at[rowv.at[0]], buf0, sg0)
        pltpu.async_copy(g_hbm.at[rowv.at[1]], buf1, sg1)

        def body(t, carry):
            j0 = 2 * t
            j1 = j0 + 1
            pltpu.make_async_copy(g_hbm.at[rowv.at[j0]], buf0, sg0).wait()
            pltpu.async_copy(buf0, acc.at[colv.at[j0]], ss0, add=True)
            pltpu.make_async_copy(g_hbm.at[rowv.at[j1]], buf1, sg1).wait()
            pltpu.async_copy(buf1, acc.at[colv.at[j1]], ss1, add=True)

            @pl.when(t + 1 < NH // 2)
            def _():
                pltpu.make_async_copy(buf0, acc.at[colv.at[j0]], ss0).wait()
                pltpu.async_copy(g_hbm.at[rowv.at[j0 + 2]], buf0, sg0)
                pltpu.make_async_copy(buf1, acc.at[colv.at[j1]], ss1).wait()
                pltpu.async_copy(g_hbm.at[rowv.at[j1 + 2]], buf1, sg1)

            return carry

        lax.fori_loop(0, NH // 2, body, 0)
        pltpu.make_async_copy(buf0, acc.at[colv.at[NH - 2]], ss0).wait()
        pltpu.make_async_copy(buf1, acc.at[colv.at[NH - 1]], ss1).wait()
    plsc.subcore_barrier()
    pltpu.sync_copy(acc.at[pl.ds(s * ZR, ZR)],
                    out_hbm.at[c, pl.ds(s * ZR, ZR), :])


def _dense1_body(deg_ref, x_ref, w_ref, g_ref, dis_ref):
    deg = deg_ref[0] + deg_ref[1] + 1.0
    dis = lax.rsqrt(deg)
    dis_b = jnp.broadcast_to(dis, (BR, C))
    h = lax.dot_general(x_ref[...], w_ref[...], (((1,), (1,)), ((), ())),
                        preferred_element_type=jnp.float32)
    g_ref[...] = dis_b * h
    dis_ref[...] = dis_b


def _dense1(degp, xp, W1):
    return pl.pallas_call(
        _dense1_body,
        grid=(NPAD // BR,),
        in_specs=[
            pl.BlockSpec((NC, BR, 1), lambda i: (0, i, 0)),
            pl.BlockSpec((BR, C), lambda i: (i, 0)),
            pl.BlockSpec((C, C), lambda i: (0, 0)),
        ],
        out_specs=[
            pl.BlockSpec((BR, C), lambda i: (i, 0)),
            pl.BlockSpec((BR, C), lambda i: (i, 0)),
        ],
        out_shape=[jax.ShapeDtypeStruct((NPAD, C), jnp.float32)] * 2,
    )(degp, xp, W1)


def _dense2_body(p_ref, g_ref, dis_ref, w_ref, b_ref, out_ref):
    t = dis_ref[...] * (p_ref[0] + p_ref[1] + g_ref[...]) + b_ref[...]
    t = jnp.maximum(t, 0.0)
    h = lax.dot_general(t, w_ref[...], (((1,), (1,)), ((), ())),
                        preferred_element_type=jnp.float32)
    out_ref[...] = dis_ref[...] * h


def _dense2(p, g, disb, W2, b1):
    return pl.pallas_call(
        _dense2_body,
        grid=(NPAD // BR,),
        in_specs=[
            pl.BlockSpec((NC, BR, C), lambda i: (0, i, 0)),
            pl.BlockSpec((BR, C), lambda i: (i, 0)),
            pl.BlockSpec((BR, C), lambda i: (i, 0)),
            pl.BlockSpec((C, C), lambda i: (0, 0)),
            pl.BlockSpec((1, C), lambda i: (0, 0)),
        ],
        out_specs=pl.BlockSpec((BR, C), lambda i: (i, 0)),
        out_shape=jax.ShapeDtypeStruct((NPAD, C), jnp.float32),
    )(p, g, disb, W2, b1)


def _dense3_body(p_ref, g_ref, dis_ref, w_ref, b2_ref, b3_ref, out_ref):
    t = dis_ref[...] * (p_ref[0] + p_ref[1] + g_ref[...]) + b2_ref[...]
    t = jnp.maximum(t, 0.0)
    z = lax.dot_general(t, w_ref[...], (((1,), (1,)), ((), ())),
                        preferred_element_type=jnp.float32) + b3_ref[...]
    m = jnp.max(z, axis=1, keepdims=True)
    e = jnp.exp(z - m)
    ssum = jnp.sum(e, axis=1, keepdims=True)
    out_ref[...] = z - m - jnp.log(ssum)


def _dense3(p, g, disb, W3, b2, b3):
    return pl.pallas_call(
        _dense3_body,
        grid=(NPAD // BR,),
        in_specs=[
            pl.BlockSpec((NC, BR, C), lambda i: (0, i, 0)),
            pl.BlockSpec((BR, C), lambda i: (i, 0)),
            pl.BlockSpec((BR, C), lambda i: (i, 0)),
            pl.BlockSpec((C, C), lambda i: (0, 0)),
            pl.BlockSpec((1, C), lambda i: (0, 0)),
            pl.BlockSpec((1, C), lambda i: (0, 0)),
        ],
        out_specs=pl.BlockSpec((BR, C), lambda i: (i, 0)),
        out_shape=jax.ShapeDtypeStruct((NPAD, C), jnp.float32),
    )(p, g, disb, W3, b2, b3)


def kernel(x, edge_index, W1, b1, W2, b2, W3, b3):
    padE = EPAD - E
    pad_idx = jnp.full((padE,), N, jnp.int32)
    rowp = jnp.concatenate([edge_index[0], pad_idx]).reshape(NW, NCH, CH)
    colp = jnp.concatenate([edge_index[1], pad_idx]).reshape(NW, NCH, CH)
    xp = jnp.pad(x, ((0, NPAD - N), (0, 0)))
    zero1 = jnp.zeros((ZR,), jnp.float32)
    zero2 = jnp.zeros((ZR, C), jnp.float32)

    deg_kernel, scatter_kernel = _sc_kernels()
    degp = deg_kernel(colp, zero1)
    g1, disb = _dense1(degp.reshape(NC, NPAD, 1), xp, W1)
    p1 = scatter_kernel(g1, rowp, colp, zero2)
    g2 = _dense2(p1, g1, disb, W2, b1.reshape(1, C))
    p2 = scatter_kernel(g2, rowp, colp, zero2)
    out = _dense3(p2, g2, disb, W3, b2.reshape(1, C), b3.reshape(1, C))
    return out[:N]
